# baseline (device time: 15380 ns/iter reference)
import jax
import jax.numpy as jnp
from jax import lax
from jax.experimental import pallas as pl
from jax.experimental.pallas import tpu as pltpu

N_DEV = 4
B, SQ, SKV, DH = 2, 128, 128, 64
H_PER = 4
D_MODEL = 512
CHUNK = D_MODEL // N_DEV
ROWS = B * SQ


def kernel(x, Wq, K_ext, V_ext, Wo):
    X2 = x.reshape(ROWS, D_MODEL)

    def body(x_ref, wq_ref, k_ref, v_ref, wo_ref, out_ref,
             x_vmem, kv_vmem,
             rs_send, rs_recv, ag_send, ag_recv,
             x_sem, kv_sems,
             rs_send_sems, rs_recv_sems, ag_send_sems, ag_recv_sems):
        me = lax.axis_index("i")

        x_dma = pltpu.make_async_copy(x_ref, x_vmem, x_sem)
        x_dma.start()
        kv_dmas = [[], []]
        for b in range(B):
            for h in range(H_PER):
                hidx = me * H_PER + h
                kd = pltpu.make_async_copy(
                    k_ref.at[b, :, hidx, :], kv_vmem.at[0, b, h],
                    kv_sems.at[b, 2 * h])
                vd = pltpu.make_async_copy(
                    v_ref.at[b, :, hidx, :], kv_vmem.at[1, b, h],
                    kv_sems.at[b, 2 * h + 1])
                kd.start()
                vd.start()
                kv_dmas[b] += [kd, vd]

        barrier_sem = pltpu.get_barrier_semaphore()
        for rel in range(1, N_DEV):
            peer = lax.rem(me + rel, N_DEV)
            pl.semaphore_signal(
                barrier_sem, inc=1,
                device_id=(peer,), device_id_type=pl.DeviceIdType.MESH,
            )

        x_dma.wait()
        q = jnp.dot(x_vmem[...], wq_ref[...],
                    preferred_element_type=jnp.float32)

        def attn(b):
            for d in kv_dmas[b]:
                d.wait()
            parts = []
            for h in range(H_PER):
                qh = q[b * SQ:(b + 1) * SQ, h * DH:(h + 1) * DH]
                kh = kv_vmem[0, b, h]
                vh = kv_vmem[1, b, h]
                s = lax.dot_general(
                    qh, kh, (((1,), (1,)), ((), ())),
                    preferred_element_type=jnp.float32) * 0.125
                w = jnp.exp(s)
                inv = 1.0 / jnp.sum(w, axis=1, keepdims=True)
                ctx_h = jnp.dot(w, vh, preferred_element_type=jnp.float32)
                parts.append(ctx_h * inv)
            return jnp.concatenate(parts, axis=1)

        dests = [lax.rem(me + rel, N_DEV) for rel in range(1, N_DEV)]
        wo_cols = [wo_ref[:, pl.ds(d * CHUNK, CHUNK)] for d in dests]
        wo_own = wo_ref[:, pl.ds(me * CHUNK, CHUNK)]

        ctx0 = attn(0)
        pl.semaphore_wait(barrier_sem, N_DEV - 1)

        rs_descs = [[], []]
        ag_descs = [[], []]
        own = [None, None]

        def rs_round(r, ctx_r):
            for j, d in enumerate(dests):
                rs_send[r, j] = jnp.dot(
                    ctx_r, wo_cols[j],
                    preferred_element_type=jnp.float32).astype(jnp.bfloat16)
                rd = pltpu.make_async_remote_copy(
                    src_ref=rs_send.at[r, j],
                    dst_ref=rs_recv.at[r, j],
                    send_sem=rs_send_sems.at[r, j],
                    recv_sem=rs_recv_sems.at[r, j],
                    device_id=(d,),
                    device_id_type=pl.DeviceIdType.MESH,
                )
                rd.start()
                rs_descs[r].append(rd)
            own[r] = jnp.dot(ctx_r, wo_own, preferred_element_type=jnp.float32)

        def reduce_and_ag(r):
            for rd in rs_descs[r]:
                rd.wait_recv()
            red = (own[r]
                   + rs_recv[r, 0].astype(jnp.float32)
                   + rs_recv[r, 1].astype(jnp.float32)
                   + rs_recv[r, 2].astype(jnp.float32))
            ag_send[r] = red.astype(jnp.bfloat16)
            out_ref[r * SQ:(r + 1) * SQ, pl.ds(me * CHUNK, CHUNK)] = red
            for j, d in enumerate(dests):
                rd = pltpu.make_async_remote_copy(
                    src_ref=ag_send.at[r],
                    dst_ref=ag_recv.at[r, j],
                    send_sem=ag_send_sems.at[r, j],
                    recv_sem=ag_recv_sems.at[r, j],
                    device_id=(d,),
                    device_id_type=pl.DeviceIdType.MESH,
                )
                rd.start()
                ag_descs[r].append(rd)

        rs_round(0, ctx0)
        ctx1 = attn(1)
        rs_round(1, ctx1)
        reduce_and_ag(0)
        reduce_and_ag(1)

        for r in range(B):
            for j in range(N_DEV - 1):
                ag_descs[r][j].wait_recv()
                src = lax.rem(me + N_DEV - 1 - j, N_DEV)
                out_ref[r * SQ:(r + 1) * SQ, pl.ds(src * CHUNK, CHUNK)] = \
                    ag_recv[r, j].astype(jnp.float32)

        for r in range(B):
            for rd in rs_descs[r] + ag_descs[r]:
                rd.wait_send()

    out2 = pl.pallas_call(
        body,
        out_shape=jax.ShapeDtypeStruct((ROWS, D_MODEL), jnp.float32),
        in_specs=[
            pl.BlockSpec(memory_space=pl.ANY),
            pl.BlockSpec(memory_space=pltpu.VMEM),
            pl.BlockSpec(memory_space=pl.ANY),
            pl.BlockSpec(memory_space=pl.ANY),
            pl.BlockSpec(memory_space=pltpu.VMEM),
        ],
        out_specs=pl.BlockSpec(memory_space=pltpu.VMEM),
        scratch_shapes=[
            pltpu.VMEM((ROWS, D_MODEL), jnp.float32),
            pltpu.VMEM((2, B, H_PER, SKV, DH), jnp.float32),
            pltpu.VMEM((B, N_DEV - 1, SQ, CHUNK), jnp.bfloat16),
            pltpu.VMEM((B, N_DEV - 1, SQ, CHUNK), jnp.bfloat16),
            pltpu.VMEM((B, SQ, CHUNK), jnp.bfloat16),
            pltpu.VMEM((B, N_DEV - 1, SQ, CHUNK), jnp.bfloat16),
            pltpu.SemaphoreType.DMA,
            pltpu.SemaphoreType.DMA((B, 2 * H_PER)),
            pltpu.SemaphoreType.DMA((B, N_DEV - 1)),
            pltpu.SemaphoreType.DMA((B, N_DEV - 1)),
            pltpu.SemaphoreType.DMA((B, N_DEV - 1)),
            pltpu.SemaphoreType.DMA((B, N_DEV - 1)),
        ],
        compiler_params=pltpu.CompilerParams(collective_id=0),
    )(X2, Wq, K_ext, V_ext, Wo)
    return out2.reshape(B, SQ, D_MODEL)


# device time: 13390 ns/iter; 1.1486x vs baseline; 1.1486x over previous
import jax
import jax.numpy as jnp
from jax import lax
from jax.experimental import pallas as pl
from jax.experimental.pallas import tpu as pltpu

N_DEV = 4
B, SQ, SKV, DH = 2, 128, 128, 64
H_PER = 4
D_MODEL = 512
CHUNK = D_MODEL // N_DEV
ROWS = B * SQ


def kernel(x, Wq, K_ext, V_ext, Wo):
    X2 = x.reshape(ROWS, D_MODEL)
    me_out = lax.axis_index("i")
    K2 = lax.dynamic_slice_in_dim(K_ext, me_out * H_PER, H_PER, axis=2)
    K2 = K2.reshape(B, SKV, H_PER * DH).astype(jnp.bfloat16)
    V2 = lax.dynamic_slice_in_dim(V_ext, me_out * H_PER, H_PER, axis=2)
    V2 = V2.reshape(B, SKV, H_PER * DH).astype(jnp.bfloat16)

    def body(x_ref, wq_ref, k_ref, v_ref, wo_ref, out_ref,
             x_vmem,
             rs_send, rs_recv, ag_send, ag_recv,
             x_sem,
             rs_send_sems, rs_recv_sems, ag_send_sems, ag_recv_sems):
        me = lax.axis_index("i")

        x_dma = pltpu.make_async_copy(x_ref, x_vmem, x_sem)
        x_dma.start()

        barrier_sem = pltpu.get_barrier_semaphore()
        for rel in range(1, N_DEV):
            peer = lax.rem(me + rel, N_DEV)
            pl.semaphore_signal(
                barrier_sem, inc=1,
                device_id=(peer,), device_id_type=pl.DeviceIdType.MESH,
            )

        x_dma.wait()
        q = jnp.dot(x_vmem[...], wq_ref[...],
                    preferred_element_type=jnp.float32)

        def attn(b):
            parts = []
            for h in range(H_PER):
                qh = q[b * SQ:(b + 1) * SQ, h * DH:(h + 1) * DH]
                kh = k_ref[b, :, h * DH:(h + 1) * DH]
                vh = v_ref[b, :, h * DH:(h + 1) * DH]
                s = lax.dot_general(
                    qh.astype(jnp.bfloat16), kh, (((1,), (1,)), ((), ())),
                    preferred_element_type=jnp.float32) * 0.125
                w = jnp.exp(s)
                inv = 1.0 / jnp.sum(w, axis=1, keepdims=True)
                ctx_h = jnp.dot(w.astype(jnp.bfloat16), vh,
                                preferred_element_type=jnp.float32)
                parts.append(ctx_h * inv)
            return jnp.concatenate(parts, axis=1)

        dests = [lax.rem(me + rel, N_DEV) for rel in range(1, N_DEV)]
        wo_cols = [wo_ref[:, pl.ds(d * CHUNK, CHUNK)] for d in dests]
        wo_own = wo_ref[:, pl.ds(me * CHUNK, CHUNK)]

        ctx0 = attn(0)
        pl.semaphore_wait(barrier_sem, N_DEV - 1)

        rs_descs = [[], []]
        ag_descs = [[], []]
        own = [None, None]

        def rs_round(r, ctx_r):
            for j, d in enumerate(dests):
                rs_send[r, j] = jnp.dot(
                    ctx_r, wo_cols[j],
                    preferred_element_type=jnp.float32).astype(jnp.bfloat16)
                rd = pltpu.make_async_remote_copy(
                    src_ref=rs_send.at[r, j],
                    dst_ref=rs_recv.at[r, j],
                    send_sem=rs_send_sems.at[r, j],
                    recv_sem=rs_recv_sems.at[r, j],
                    device_id=(d,),
                    device_id_type=pl.DeviceIdType.MESH,
                )
                rd.start()
                rs_descs[r].append(rd)
            own[r] = jnp.dot(ctx_r, wo_own, preferred_element_type=jnp.float32)

        def reduce_and_ag(r):
            for rd in rs_descs[r]:
                rd.wait_recv()
            red = (own[r]
                   + rs_recv[r, 0].astype(jnp.float32)
                   + rs_recv[r, 1].astype(jnp.float32)
                   + rs_recv[r, 2].astype(jnp.float32))
            ag_send[r] = red.astype(jnp.bfloat16)
            out_ref[r * SQ:(r + 1) * SQ, pl.ds(me * CHUNK, CHUNK)] = red
            for j, d in enumerate(dests):
                rd = pltpu.make_async_remote_copy(
                    src_ref=ag_send.at[r],
                    dst_ref=ag_recv.at[r, j],
                    send_sem=ag_send_sems.at[r, j],
                    recv_sem=ag_recv_sems.at[r, j],
                    device_id=(d,),
                    device_id_type=pl.DeviceIdType.MESH,
                )
                rd.start()
                ag_descs[r].append(rd)

        rs_round(0, ctx0)
        ctx1 = attn(1)
        rs_round(1, ctx1)
        reduce_and_ag(0)
        reduce_and_ag(1)

        for r in range(B):
            for j in range(N_DEV - 1):
                ag_descs[r][j].wait_recv()
                src = lax.rem(me + N_DEV - 1 - j, N_DEV)
                out_ref[r * SQ:(r + 1) * SQ, pl.ds(src * CHUNK, CHUNK)] = \
                    ag_recv[r, j].astype(jnp.float32)

        for r in range(B):
            for rd in rs_descs[r] + ag_descs[r]:
                rd.wait_send()

    out2 = pl.pallas_call(
        body,
        out_shape=jax.ShapeDtypeStruct((ROWS, D_MODEL), jnp.float32),
        in_specs=[
            pl.BlockSpec(memory_space=pl.ANY),
            pl.BlockSpec(memory_space=pltpu.VMEM),
            pl.BlockSpec(memory_space=pltpu.VMEM),
            pl.BlockSpec(memory_space=pltpu.VMEM),
            pl.BlockSpec(memory_space=pltpu.VMEM),
        ],
        out_specs=pl.BlockSpec(memory_space=pltpu.VMEM),
        scratch_shapes=[
            pltpu.VMEM((ROWS, D_MODEL), jnp.float32),
            pltpu.VMEM((B, N_DEV - 1, SQ, CHUNK), jnp.bfloat16),
            pltpu.VMEM((B, N_DEV - 1, SQ, CHUNK), jnp.bfloat16),
            pltpu.VMEM((B, SQ, CHUNK), jnp.bfloat16),
            pltpu.VMEM((B, N_DEV - 1, SQ, CHUNK), jnp.bfloat16),
            pltpu.SemaphoreType.DMA,
            pltpu.SemaphoreType.DMA((B, N_DEV - 1)),
            pltpu.SemaphoreType.DMA((B, N_DEV - 1)),
            pltpu.SemaphoreType.DMA((B, N_DEV - 1)),
            pltpu.SemaphoreType.DMA((B, N_DEV - 1)),
        ],
        compiler_params=pltpu.CompilerParams(collective_id=0),
    )(X2, Wq, K2, V2, Wo)
    return out2.reshape(B, SQ, D_MODEL)


# device time: 13366 ns/iter; 1.1507x vs baseline; 1.0018x over previous
import jax
import jax.numpy as jnp
from jax import lax
from jax.experimental import pallas as pl
from jax.experimental.pallas import tpu as pltpu

N_DEV = 4
B, SQ, SKV, DH = 2, 128, 128, 64
H_PER = 4
D_MODEL = 512
CHUNK = D_MODEL // N_DEV
ROWS = B * SQ


def kernel(x, Wq, K_ext, V_ext, Wo):
    X2 = x.reshape(ROWS, D_MODEL)
    me_out = lax.axis_index("i")
    K2 = lax.dynamic_slice_in_dim(K_ext, me_out * H_PER, H_PER, axis=2)
    K2 = K2.reshape(B, SKV, H_PER * DH).astype(jnp.bfloat16)
    V2 = lax.dynamic_slice_in_dim(V_ext, me_out * H_PER, H_PER, axis=2)
    V2 = V2.reshape(B, SKV, H_PER * DH).astype(jnp.bfloat16)

    def body(x_ref, wq_ref, k_ref, v_ref, wo_ref, out_ref,
             x_vmem,
             rs_send, rs_recv, ag_send, ag_recv,
             x_sem,
             rs_send_sems, rs_recv_sems, ag_send_sems, ag_recv_sems):
        me = lax.axis_index("i")

        x_dma = pltpu.make_async_copy(x_ref, x_vmem, x_sem)
        x_dma.start()

        barrier_sem = pltpu.get_barrier_semaphore()
        for rel in range(1, N_DEV):
            peer = lax.rem(me + rel, N_DEV)
            pl.semaphore_signal(
                barrier_sem, inc=1,
                device_id=(peer,), device_id_type=pl.DeviceIdType.MESH,
            )

        x_dma.wait()
        q = jnp.dot(x_vmem[...], wq_ref[...],
                    preferred_element_type=jnp.float32)

        def attn(b):
            parts = []
            for h in range(H_PER):
                qh = q[b * SQ:(b + 1) * SQ, h * DH:(h + 1) * DH]
                kh = k_ref[b, :, h * DH:(h + 1) * DH]
                vh = v_ref[b, :, h * DH:(h + 1) * DH]
                s = lax.dot_general(
                    qh.astype(jnp.bfloat16), kh, (((1,), (1,)), ((), ())),
                    preferred_element_type=jnp.float32) * 0.125
                w = jnp.exp(s)
                inv = 1.0 / jnp.sum(w, axis=1, keepdims=True)
                ctx_h = jnp.dot(w.astype(jnp.bfloat16), vh,
                                preferred_element_type=jnp.float32)
                parts.append(ctx_h * inv)
            return jnp.concatenate(parts, axis=1)

        dests = [lax.rem(me + rel, N_DEV) for rel in range(1, N_DEV)]
        wo_cols = [wo_ref[:, pl.ds(d * CHUNK, CHUNK)] for d in dests]
        wo_own = wo_ref[:, pl.ds(me * CHUNK, CHUNK)]

        ctx0 = attn(0)
        pl.semaphore_wait(barrier_sem, N_DEV - 1)

        rs_descs = [[], []]
        ag_descs = [[], []]
        own = [None, None]

        def rs_round(r, ctx_r):
            for j, d in enumerate(dests):
                rs_send[r, j] = jnp.dot(
                    ctx_r, wo_cols[j],
                    preferred_element_type=jnp.float32).astype(jnp.bfloat16)
                rd = pltpu.make_async_remote_copy(
                    src_ref=rs_send.at[r, j],
                    dst_ref=rs_recv.at[r, j],
                    send_sem=rs_send_sems.at[r, j],
                    recv_sem=rs_recv_sems.at[r, j],
                    device_id=(d,),
                    device_id_type=pl.DeviceIdType.MESH,
                )
                rd.start()
                rs_descs[r].append(rd)
            own[r] = jnp.dot(ctx_r, wo_own, preferred_element_type=jnp.float32)

        def reduce_and_ag(r):
            for rd in rs_descs[r]:
                rd.wait_recv()
            red = (own[r]
                   + rs_recv[r, 0].astype(jnp.float32)
                   + rs_recv[r, 1].astype(jnp.float32)
                   + rs_recv[r, 2].astype(jnp.float32))
            ag_send[r] = red.astype(jnp.bfloat16)
            out_ref[r * SQ:(r + 1) * SQ, pl.ds(me * CHUNK, CHUNK)] = red
            for j, d in enumerate(dests):
                rd = pltpu.make_async_remote_copy(
                    src_ref=ag_send.at[r],
                    dst_ref=ag_recv.at[r, j],
                    send_sem=ag_send_sems.at[r, j],
                    recv_sem=ag_recv_sems.at[r, j],
                    device_id=(d,),
                    device_id_type=pl.DeviceIdType.MESH,
                )
                rd.start()
                ag_descs[r].append(rd)

        rs_round(0, ctx0)
        ctx1 = attn(1)
        rs_round(1, ctx1)
        reduce_and_ag(0)
        reduce_and_ag(1)

        for r in range(B):
            for j in range(N_DEV - 1):
                ag_descs[r][j].wait_recv()
                src = lax.rem(me + N_DEV - 1 - j, N_DEV)
                out_ref[r * SQ:(r + 1) * SQ, pl.ds(src * CHUNK, CHUNK)] = \
                    ag_recv[r, j].astype(jnp.float32)

        for r in range(B):
            for rd in rs_descs[r] + ag_descs[r]:
                rd.wait_send()

    out2 = pl.pallas_call(
        body,
        out_shape=jax.ShapeDtypeStruct((ROWS, D_MODEL), jnp.float32),
        in_specs=[
            pl.BlockSpec(memory_space=pltpu.MemorySpace.HBM),
            pl.BlockSpec(memory_space=pltpu.VMEM),
            pl.BlockSpec(memory_space=pltpu.VMEM),
            pl.BlockSpec(memory_space=pltpu.VMEM),
            pl.BlockSpec(memory_space=pltpu.VMEM),
        ],
        out_specs=pl.BlockSpec(memory_space=pltpu.VMEM),
        scratch_shapes=[
            pltpu.VMEM((ROWS, D_MODEL), jnp.float32),
            pltpu.VMEM((B, N_DEV - 1, SQ, CHUNK), jnp.bfloat16),
            pltpu.VMEM((B, N_DEV - 1, SQ, CHUNK), jnp.bfloat16),
            pltpu.VMEM((B, SQ, CHUNK), jnp.bfloat16),
            pltpu.VMEM((B, N_DEV - 1, SQ, CHUNK), jnp.bfloat16),
            pltpu.SemaphoreType.DMA,
            pltpu.SemaphoreType.DMA((B, N_DEV - 1)),
            pltpu.SemaphoreType.DMA((B, N_DEV - 1)),
            pltpu.SemaphoreType.DMA((B, N_DEV - 1)),
            pltpu.SemaphoreType.DMA((B, N_DEV - 1)),
        ],
        compiler_params=pltpu.CompilerParams(collective_id=0),
    )(X2, Wq, K2, V2, Wo)
    return out2.reshape(B, SQ, D_MODEL)


# device time: 12924 ns/iter; 1.1900x vs baseline; 1.0342x over previous
import jax
import jax.numpy as jnp
from jax import lax
from jax.experimental import pallas as pl
from jax.experimental.pallas import tpu as pltpu

N_DEV = 4
B, SQ, SKV, DH = 2, 128, 128, 64
H_PER = 4
D_MODEL = 512
CHUNK = D_MODEL // N_DEV
ROWS = B * SQ


def kernel(x, Wq, K_ext, V_ext, Wo):
    X2 = x.reshape(ROWS, D_MODEL)
    X2 = pltpu.with_memory_space_constraint(X2, pltpu.MemorySpace.HBM)
    me_out = lax.axis_index("i")
    K2 = lax.dynamic_slice_in_dim(K_ext, me_out * H_PER, H_PER, axis=2)
    K2 = K2.reshape(B, SKV, H_PER * DH).astype(jnp.bfloat16)
    V2 = lax.dynamic_slice_in_dim(V_ext, me_out * H_PER, H_PER, axis=2)
    V2 = V2.reshape(B, SKV, H_PER * DH).astype(jnp.bfloat16)

    def body(x_ref, wq_ref, k_ref, v_ref, wo_ref, out_ref,
             x_vmem,
             rs_send, rs_recv, ag_send, ag_recv,
             x_sem,
             rs_send_sems, rs_recv_sems, ag_send_sems, ag_recv_sems):
        me = lax.axis_index("i")

        x_dma = pltpu.make_async_copy(x_ref, x_vmem, x_sem)
        x_dma.start()

        barrier_sem = pltpu.get_barrier_semaphore()
        for rel in range(1, N_DEV):
            peer = lax.rem(me + rel, N_DEV)
            pl.semaphore_signal(
                barrier_sem, inc=1,
                device_id=(peer,), device_id_type=pl.DeviceIdType.MESH,
            )

        x_dma.wait()
        q = jnp.dot(x_vmem[...], wq_ref[...],
                    preferred_element_type=jnp.float32)

        def attn(b):
            parts = []
            for h in range(H_PER):
                qh = q[b * SQ:(b + 1) * SQ, h * DH:(h + 1) * DH]
                kh = k_ref[b, :, h * DH:(h + 1) * DH]
                vh = v_ref[b, :, h * DH:(h + 1) * DH]
                s = lax.dot_general(
                    qh.astype(jnp.bfloat16), kh, (((1,), (1,)), ((), ())),
                    preferred_element_type=jnp.float32) * 0.125
                w = jnp.exp(s)
                inv = 1.0 / jnp.sum(w, axis=1, keepdims=True)
                ctx_h = jnp.dot(w.astype(jnp.bfloat16), vh,
                                preferred_element_type=jnp.float32)
                parts.append(ctx_h * inv)
            return jnp.concatenate(parts, axis=1)

        dests = [lax.rem(me + rel, N_DEV) for rel in range(1, N_DEV)]
        wo_cols = [wo_ref[:, pl.ds(d * CHUNK, CHUNK)] for d in dests]
        wo_own = wo_ref[:, pl.ds(me * CHUNK, CHUNK)]

        ctx0 = attn(0)
        pl.semaphore_wait(barrier_sem, N_DEV - 1)

        rs_descs = [[], []]
        ag_descs = [[], []]
        own = [None, None]

        def rs_round(r, ctx_r):
            for j, d in enumerate(dests):
                rs_send[r, j] = jnp.dot(
                    ctx_r, wo_cols[j],
                    preferred_element_type=jnp.float32).astype(jnp.bfloat16)
                rd = pltpu.make_async_remote_copy(
                    src_ref=rs_send.at[r, j],
                    dst_ref=rs_recv.at[r, j],
                    send_sem=rs_send_sems.at[r, j],
                    recv_sem=rs_recv_sems.at[r, j],
                    device_id=(d,),
                    device_id_type=pl.DeviceIdType.MESH,
                )
                rd.start()
                rs_descs[r].append(rd)
            own[r] = jnp.dot(ctx_r, wo_own, preferred_element_type=jnp.float32)

        def reduce_and_ag(r):
            for rd in rs_descs[r]:
                rd.wait_recv()
            red = (own[r]
                   + rs_recv[r, 0].astype(jnp.float32)
                   + rs_recv[r, 1].astype(jnp.float32)
                   + rs_recv[r, 2].astype(jnp.float32))
            ag_send[r] = red.astype(jnp.bfloat16)
            out_ref[r * SQ:(r + 1) * SQ, pl.ds(me * CHUNK, CHUNK)] = red
            for j, d in enumerate(dests):
                rd = pltpu.make_async_remote_copy(
                    src_ref=ag_send.at[r],
                    dst_ref=ag_recv.at[r, j],
                    send_sem=ag_send_sems.at[r, j],
                    recv_sem=ag_recv_sems.at[r, j],
                    device_id=(d,),
                    device_id_type=pl.DeviceIdType.MESH,
                )
                rd.start()
                ag_descs[r].append(rd)

        rs_round(0, ctx0)
        ctx1 = attn(1)
        rs_round(1, ctx1)
        reduce_and_ag(0)
        reduce_and_ag(1)

        for r in range(B):
            for j in range(N_DEV - 1):
                ag_descs[r][j].wait_recv()
                src = lax.rem(me + N_DEV - 1 - j, N_DEV)
                out_ref[r * SQ:(r + 1) * SQ, pl.ds(src * CHUNK, CHUNK)] = \
                    ag_recv[r, j].astype(jnp.float32)

        for r in range(B):
            for rd in rs_descs[r] + ag_descs[r]:
                rd.wait_send()

    out2 = pl.pallas_call(
        body,
        out_shape=jax.ShapeDtypeStruct((ROWS, D_MODEL), jnp.float32),
        in_specs=[
            pl.BlockSpec(memory_space=pltpu.MemorySpace.HBM),
            pl.BlockSpec(memory_space=pltpu.VMEM),
            pl.BlockSpec(memory_space=pltpu.VMEM),
            pl.BlockSpec(memory_space=pltpu.VMEM),
            pl.BlockSpec(memory_space=pltpu.VMEM),
        ],
        out_specs=pl.BlockSpec(memory_space=pltpu.VMEM),
        scratch_shapes=[
            pltpu.VMEM((ROWS, D_MODEL), jnp.float32),
            pltpu.VMEM((B, N_DEV - 1, SQ, CHUNK), jnp.bfloat16),
            pltpu.VMEM((B, N_DEV - 1, SQ, CHUNK), jnp.bfloat16),
            pltpu.VMEM((B, SQ, CHUNK), jnp.bfloat16),
            pltpu.VMEM((B, N_DEV - 1, SQ, CHUNK), jnp.bfloat16),
            pltpu.SemaphoreType.DMA,
            pltpu.SemaphoreType.DMA((B, N_DEV - 1)),
            pltpu.SemaphoreType.DMA((B, N_DEV - 1)),
            pltpu.SemaphoreType.DMA((B, N_DEV - 1)),
            pltpu.SemaphoreType.DMA((B, N_DEV - 1)),
        ],
        compiler_params=pltpu.CompilerParams(collective_id=0),
    )(X2, Wq, K2, V2, Wo)
    return out2.reshape(B, SQ, D_MODEL)
